# baseline (speedup 1.0000x reference)
import functools

import jax
import jax.numpy as jnp
from jax import lax
from jax.experimental import pallas as pl
from jax.experimental.pallas import tpu as pltpu
from jax.experimental.pallas import tpu_sc as plsc

_EPS = 1e-06
_D_OUT = 2048


def _table_body(emb_ref, nw_ref, pw_ref, out_ref):
    emb = emb_ref[...]
    normed = emb * lax.rsqrt(jnp.mean(emb * emb, axis=-1, keepdims=True) + _EPS)
    normed = normed * nw_ref[...]
    proj = lax.dot_general(
        normed, pw_ref[...], (((1,), (1,)), ((), ())),
        preferred_element_type=jnp.float32)
    out_ref[...] = proj * lax.rsqrt(
        jnp.mean(proj * proj, axis=-1, keepdims=True) + _EPS)


def _make_gather(vocab, d, batch):
    info = plsc.get_sparse_core_info()
    nc, ns = info.num_cores, info.num_subcores
    nw = nc * ns
    assert batch % (8 * nw) == 0
    b_per_w = batch // nw
    vec = 16
    n_vec = b_per_w // vec
    mesh = plsc.VectorSubcoreMesh(core_axis_name="c", subcore_axis_name="s")

    @functools.partial(
        pl.kernel,
        mesh=mesh,
        out_type=jax.ShapeDtypeStruct((batch, d), jnp.float32),
        scratch_types=[
            pltpu.VMEM((b_per_w,), jnp.int32),
            pltpu.SemaphoreType.DMA,
        ],
    )
    def gather(table_hbm, ids_hbm, out_hbm, idx_v, sem):
        wid = lax.axis_index("s") * nc + lax.axis_index("c")
        base = wid * b_per_w
        pltpu.sync_copy(ids_hbm.at[pl.ds(base, b_per_w)], idx_v)

        def start_row(row_id, dst_i):
            pltpu.async_copy(table_hbm.at[row_id], out_hbm.at[dst_i], sem)

        def wait_row():
            pltpu.make_async_copy(table_hbm.at[0], out_hbm.at[0], sem).wait()

        def issue16(j, wait_first):
            ids16 = idx_v[pl.ds(j * vec, vec)]
            for k in range(vec):
                if wait_first:
                    wait_row()
                start_row(ids16[k], base + j * vec + k)

        issue16(0, False)

        def body(j, _):
            issue16(j, True)
            return 0

        lax.fori_loop(1, n_vec, body, 0)
        lax.fori_loop(0, vec, lambda i, _: (wait_row(), 0)[1], 0)

    return gather


def kernel(input_ids, embedding, hard_norm_weight, proj_weight):
    vocab, mm_hidden = embedding.shape
    b, s = input_ids.shape
    table = pl.pallas_call(
        _table_body,
        out_shape=jax.ShapeDtypeStruct((vocab, _D_OUT), jnp.float32),
    )(embedding, hard_norm_weight.reshape(1, mm_hidden), proj_weight)
    n_workers = 32
    table_rep = jnp.broadcast_to(
        table, (n_workers, vocab, _D_OUT)).reshape(n_workers * vocab, _D_OUT)
    ids_flat = input_ids.reshape(b * s).astype(jnp.int32)
    per_w = (b * s) // n_workers
    ids_flat = ids_flat + jnp.repeat(
        jnp.arange(n_workers, dtype=jnp.int32) * vocab, per_w)
    gather = _make_gather(n_workers * vocab, _D_OUT, b * s)
    out = gather(table_rep, ids_flat)
    return out.reshape(b, s, _D_OUT)


# speedup vs baseline: 36.0401x; 36.0401x over previous
import functools

import jax
import jax.numpy as jnp
from jax import lax
from jax.experimental import pallas as pl
from jax.experimental.pallas import tpu as pltpu
from jax.experimental.pallas import tpu_sc as plsc

_EPS = 1e-06
_D_OUT = 2048


def _table_body(emb_ref, nw_ref, pw_ref, out_ref):
    emb = emb_ref[...]
    normed = emb * lax.rsqrt(jnp.mean(emb * emb, axis=-1, keepdims=True) + _EPS)
    normed = normed * nw_ref[...]
    proj = lax.dot_general(
        normed, pw_ref[...], (((1,), (1,)), ((), ())),
        preferred_element_type=jnp.float32)
    out_ref[...] = proj * lax.rsqrt(
        jnp.mean(proj * proj, axis=-1, keepdims=True) + _EPS)


def _make_gather(vocab, d, batch):
    info = plsc.get_sparse_core_info()
    nc, ns = info.num_cores, info.num_subcores
    nw = nc * ns
    assert batch % (8 * nw) == 0
    b_per_w = batch // nw
    chunk = 8
    nbuf = 4
    n_chunks = b_per_w // chunk
    assert n_chunks % nbuf == 0 and (n_chunks - 4) % nbuf == 0
    mesh = plsc.VectorSubcoreMesh(core_axis_name="c", subcore_axis_name="s")

    @functools.partial(
        pl.kernel,
        mesh=mesh,
        out_type=jax.ShapeDtypeStruct((batch, d), jnp.float32),
        scratch_types=(
            [pltpu.VMEM((b_per_w,), jnp.int32)]
            + [pltpu.VMEM((chunk, d), jnp.float32)] * nbuf
            + [pltpu.SemaphoreType.DMA] * (2 * nbuf)
        ),
    )
    def gather(table_hbm, ids_hbm, out_hbm, idx_v, *bufsem):
        bufs = bufsem[:nbuf]
        gsems = bufsem[nbuf:2 * nbuf]
        wsems = bufsem[2 * nbuf:]
        wid = lax.axis_index("s") * nc + lax.axis_index("c")
        base = wid * b_per_w
        pltpu.sync_copy(ids_hbm.at[pl.ds(base, b_per_w)], idx_v)

        def start_gather(i, b):
            pltpu.async_copy(
                table_hbm.at[idx_v.at[pl.ds(i * chunk, chunk)]],
                bufs[b], gsems[b])

        def wait_gather(b):
            pltpu.make_async_copy(
                table_hbm.at[idx_v.at[pl.ds(0, chunk)]],
                bufs[b], gsems[b]).wait()

        def start_write(i, b):
            pltpu.async_copy(
                bufs[b], out_hbm.at[pl.ds(base + i * chunk, chunk)], wsems[b])

        def wait_write(b):
            pltpu.make_async_copy(
                bufs[b], out_hbm.at[pl.ds(base, chunk)], wsems[b]).wait()

        start_gather(0, 0)
        start_gather(1, 1)
        for i in (0, 1):
            wait_gather(i)
            start_write(i, i)
            start_gather(i + 2, i + 2)

        def body(p, _):
            for par in range(nbuf):
                i = p * nbuf + par + 2
                b = (par + 2) % nbuf
                wait_gather(b)
                start_write(i, b)
                wait_write(par)
                start_gather(i + 2, par)
            return 0

        lax.fori_loop(0, (n_chunks - 4) // nbuf, body, 0)

        for i in (n_chunks - 2, n_chunks - 1):
            b = i % nbuf
            wait_gather(b)
            start_write(i, b)
        for b in range(nbuf):
            wait_write(b)

    return gather


def kernel(input_ids, embedding, hard_norm_weight, proj_weight):
    vocab, mm_hidden = embedding.shape
    b, s = input_ids.shape
    table = pl.pallas_call(
        _table_body,
        out_shape=jax.ShapeDtypeStruct((vocab, _D_OUT), jnp.float32),
    )(embedding, hard_norm_weight.reshape(1, mm_hidden), proj_weight)
    n_workers = 32
    table_rep = jnp.broadcast_to(
        table, (n_workers, vocab, _D_OUT)).reshape(n_workers * vocab, _D_OUT)
    ids_flat = input_ids.reshape(b * s).astype(jnp.int32)
    per_w = (b * s) // n_workers
    ids_flat = ids_flat + jnp.repeat(
        jnp.arange(n_workers, dtype=jnp.int32) * vocab, per_w)
    gather = _make_gather(n_workers * vocab, _D_OUT, b * s)
    out = gather(table_rep, ids_flat)
    return out.reshape(b, s, _D_OUT)


# speedup vs baseline: 80.8551x; 2.2435x over previous
import functools

import jax
import jax.numpy as jnp
from jax import lax
from jax.experimental import pallas as pl
from jax.experimental.pallas import tpu as pltpu
from jax.experimental.pallas import tpu_sc as plsc

_EPS = 1e-06
_D_OUT = 2048


def _table_body(emb_ref, nw_ref, pw_ref, out_ref):
    emb = emb_ref[...]
    normed = emb * lax.rsqrt(jnp.mean(emb * emb, axis=-1, keepdims=True) + _EPS)
    normed = normed * nw_ref[...]
    proj = lax.dot_general(
        normed, pw_ref[...], (((1,), (1,)), ((), ())),
        preferred_element_type=jnp.float32)
    out_ref[...] = proj * lax.rsqrt(
        jnp.mean(proj * proj, axis=-1, keepdims=True) + _EPS)


def _make_gather(vocab, d, batch):
    info = plsc.get_sparse_core_info()
    nc, ns = info.num_cores, info.num_subcores
    nw = nc * ns
    assert batch % (8 * nw) == 0
    b_per_w = batch // nw
    chunk = 8
    nbuf = 4
    n_chunks = b_per_w // chunk
    assert n_chunks % nbuf == 0 and (n_chunks - 4) % nbuf == 0
    mesh = plsc.VectorSubcoreMesh(core_axis_name="c", subcore_axis_name="s")

    @functools.partial(
        pl.kernel,
        mesh=mesh,
        out_type=jax.ShapeDtypeStruct((batch, d), jnp.float32),
        scratch_types=(
            [pltpu.VMEM((b_per_w,), jnp.int32)]
            + [pltpu.VMEM((chunk, d), jnp.float32)] * nbuf
            + [pltpu.SemaphoreType.DMA] * (2 * nbuf)
        ),
    )
    def gather(table_hbm, ids_hbm, out_hbm, idx_v, *bufsem):
        bufs = bufsem[:nbuf]
        gsems = bufsem[nbuf:2 * nbuf]
        wsems = bufsem[2 * nbuf:]
        wid = lax.axis_index("s") * nc + lax.axis_index("c")
        base = wid * b_per_w
        pltpu.sync_copy(ids_hbm.at[pl.ds(base, b_per_w)], idx_v)

        def start_gather(i, b):
            pltpu.async_copy(
                table_hbm.at[idx_v.at[pl.ds(i * chunk, chunk)]],
                bufs[b], gsems[b])

        def wait_gather(b):
            pltpu.make_async_copy(
                table_hbm.at[idx_v.at[pl.ds(0, chunk)]],
                bufs[b], gsems[b]).wait()

        def start_write(i, b):
            pltpu.async_copy(
                bufs[b], out_hbm.at[pl.ds(base + i * chunk, chunk)], wsems[b])

        def wait_write(b):
            pltpu.make_async_copy(
                bufs[b], out_hbm.at[pl.ds(base, chunk)], wsems[b]).wait()

        start_gather(0, 0)
        start_gather(1, 1)
        for i in (0, 1):
            wait_gather(i)
            start_write(i, i)
            start_gather(i + 2, i + 2)

        def body(p, _):
            for par in range(nbuf):
                i = p * nbuf + par + 2
                b = (par + 2) % nbuf
                wait_gather(b)
                start_write(i, b)
                wait_write(par)
                start_gather(i + 2, par)
            return 0

        lax.fori_loop(0, (n_chunks - 4) // nbuf, body, 0)

        for i in (n_chunks - 2, n_chunks - 1):
            b = i % nbuf
            wait_gather(b)
            start_write(i, b)
        for b in range(nbuf):
            wait_write(b)

    return gather


def _onehot_gather_body(ids_ref, table_ref, out_ref):
    ids = ids_ref[0, 0]
    onehot = (ids[:, None] == lax.broadcasted_iota(
        jnp.int32, (1, table_ref.shape[0]), 1)).astype(jnp.float32)
    out_ref[...] = lax.dot_general(
        onehot, table_ref[...], (((1,), (0,)), ((), ())),
        preferred_element_type=jnp.float32)


def _tc_gather(table, ids, bt=512):
    (t,) = ids.shape
    vocab, d = table.shape
    assert t % bt == 0
    grid = t // bt
    return pl.pallas_call(
        _onehot_gather_body,
        grid=(grid,),
        in_specs=[
            pl.BlockSpec((1, 1, bt), lambda i: (i, 0, 0)),
            pl.BlockSpec((vocab, d), lambda i: (0, 0)),
        ],
        out_specs=pl.BlockSpec((bt, d), lambda i: (i, 0)),
        out_shape=jax.ShapeDtypeStruct((t, d), jnp.float32),
    )(ids.reshape(grid, 1, bt), table)


def kernel(input_ids, embedding, hard_norm_weight, proj_weight):
    vocab, mm_hidden = embedding.shape
    b, s = input_ids.shape
    table = pl.pallas_call(
        _table_body,
        out_shape=jax.ShapeDtypeStruct((vocab, _D_OUT), jnp.float32),
    )(embedding, hard_norm_weight.reshape(1, mm_hidden), proj_weight)
    ids_flat = input_ids.reshape(b * s).astype(jnp.int32)
    out = _tc_gather(table, ids_flat)
    return out.reshape(b, s, _D_OUT)
